# baseline (speedup 1.0000x reference)
import functools

import jax
import jax.numpy as jnp
from jax import lax
from jax.experimental import pallas as pl
from jax.experimental.pallas import tpu as pltpu
from jax.experimental.pallas import tpu_sc as plsc

_NUM_BET_BINS = 32
_D = 128
_NUM_STREETS = 4
_ACTION_OFFSET = 10

_N = 204800
_NW = 32
_RPW = _N // _NW
_C = 128
_NCH = _RPW // _C
_L = 16
_NJ = _D // _L

_mesh = plsc.VectorSubcoreMesh(core_axis_name="c", subcore_axis_name="s")


def _take16(v, idx):
    return lax.gather(
        v, idx[:, None],
        dimension_numbers=lax.GatherDimensionNumbers(
            offset_dims=(), collapsed_slice_dims=(0,), start_index_map=(0,)),
        slice_sizes=(1,),
        mode=lax.GatherScatterMode.PROMISE_IN_BOUNDS)


def _rsqrt16(x):
    i = lax.bitcast_convert_type(x, jnp.int32)
    i = jnp.full((_L,), 0x5F3759DF, jnp.int32) - lax.shift_right_logical(
        i, jnp.full((_L,), 1, jnp.int32))
    y = lax.bitcast_convert_type(i, jnp.float32)
    for _ in range(3):
        y = y * (1.5 - 0.5 * x * y * y)
    return y


@functools.partial(
    pl.kernel,
    mesh=_mesh,
    out_type=jax.ShapeDtypeStruct((_N, _D), jnp.float32),
    scratch_types=[
        pltpu.VMEM((2 * _C, _NUM_BET_BINS), jnp.float32),
        pltpu.VMEM((2 * _C,), jnp.int32),
        pltpu.VMEM((2 * _C,), jnp.int32),
        pltpu.VMEM((2 * _C,), jnp.int32),
        pltpu.VMEM((_NUM_BET_BINS, _D), jnp.float32),
        pltpu.VMEM((2, _D), jnp.float32),
        pltpu.VMEM((_NUM_STREETS, _D), jnp.float32),
        pltpu.VMEM((_NUM_BET_BINS, _D), jnp.float32),
        pltpu.VMEM((_D,), jnp.float32),
        pltpu.VMEM((_D,), jnp.float32),
        pltpu.VMEM((_D,), jnp.float32),
        pltpu.VMEM((256, _D), jnp.float32),
        pltpu.VMEM((2 * _C, _D), jnp.float32),
        pltpu.SemaphoreType.DMA,
        pltpu.SemaphoreType.DMA,
    ],
)
def _sc_run(masks_hbm, tok_hbm, act_hbm, st_hbm, actor_w_hbm, street_w_hbm,
            type_w_hbm, mlp_w_hbm, b_hbm, g_hbm, be_hbm, out_hbm,
            m_v, tok_v, act_v, st_v, w_v, actor_v, street_v, type_v,
            b_v, g_v, be_v, comb_v, out_v, sem_o, sem_i):
    cid = lax.axis_index("c")
    sid = lax.axis_index("s")
    wid = sid * 2 + cid
    base = wid * _RPW

    pltpu.sync_copy(mlp_w_hbm, w_v)
    pltpu.sync_copy(actor_w_hbm, actor_v)
    pltpu.sync_copy(street_w_hbm, street_v)
    pltpu.sync_copy(type_w_hbm, type_v)
    pltpu.sync_copy(b_hbm, b_v)
    pltpu.sync_copy(g_hbm, g_v)
    pltpu.sync_copy(be_hbm, be_v)

    for a in range(2):
        for s in range(_NUM_STREETS):
            base8 = [actor_v[a, pl.ds(j * _L, _L)]
                     + street_v[s, pl.ds(j * _L, _L)] for j in range(_NJ)]

            def bin_body(b, carry, _base8=base8, _row0=(a * 4 + s) * 32):
                for j in range(_NJ):
                    comb_v[_row0 + b, pl.ds(j * _L, _L)] = (
                        _base8[j] + type_v[b, pl.ds(j * _L, _L)])
                return carry

            lax.fori_loop(0, _NUM_BET_BINS, bin_body, 0)

    def _start_in(t):
        row0 = base + t * _C
        ih = (t % 2) * _C
        pltpu.async_copy(masks_hbm.at[pl.ds(row0, _C)],
                         m_v.at[pl.ds(ih, _C)], sem_i)
        pltpu.async_copy(tok_hbm.at[pl.ds(row0, _C)],
                         tok_v.at[pl.ds(ih, _C)], sem_i)
        pltpu.async_copy(act_hbm.at[pl.ds(row0, _C)],
                         act_v.at[pl.ds(ih, _C)], sem_i)
        pltpu.async_copy(st_hbm.at[pl.ds(row0, _C)],
                         st_v.at[pl.ds(ih, _C)], sem_i)

    def _wait_in():
        pltpu.make_async_copy(masks_hbm.at[pl.ds(base, _C)],
                              m_v.at[pl.ds(0, _C)], sem_i).wait()
        pltpu.make_async_copy(tok_hbm.at[pl.ds(base, _C)],
                              tok_v.at[pl.ds(0, _C)], sem_i).wait()
        pltpu.make_async_copy(act_hbm.at[pl.ds(base, _C)],
                              act_v.at[pl.ds(0, _C)], sem_i).wait()
        pltpu.make_async_copy(st_hbm.at[pl.ds(base, _C)],
                              st_v.at[pl.ds(0, _C)], sem_i).wait()

    _start_in(0)

    def chunk_body(t, carry):
        row0 = base + t * _C
        _wait_in()

        @pl.when(t + 1 < _NCH)
        def _prefetch():
            _start_in(t + 1)

        @pl.when(t >= 2)
        def _wait_prev():
            pltpu.make_async_copy(out_v.at[pl.ds(0, _C)],
                                  out_hbm.at[pl.ds(base, _C)], sem_o).wait()

        half = (t % 2) * _C
        ihalf = (t % 2) * _C

        def sg_body(sg, gcarry):
            r0 = sg * _L
            iota = lax.broadcasted_iota(jnp.int32, (_L,), 0)
            tok16 = tok_v[pl.ds(ihalf + r0, _L)]
            a16 = jnp.clip(act_v[pl.ds(ihalf + r0, _L)], 0, 1)
            s16 = jnp.clip(st_v[pl.ds(ihalf + r0, _L)],
                           0, _NUM_STREETS - 1)
            aid16 = jnp.clip(tok16 - _ACTION_OFFSET, 0, _NUM_BET_BINS - 1)
            cix16 = (a16 * 4 + s16) * 32 + aid16
            valid16 = jnp.where(
                (tok16 >= _ACTION_OFFSET)
                & (tok16 < _ACTION_OFFSET + _NUM_BET_BINS),
                jnp.full((_L,), 1.0, jnp.float32),
                jnp.full((_L,), 0.0, jnp.float32))
            bias8 = [b_v[pl.ds(j * _L, _L)] for j in range(_NJ)]
            gam8 = [g_v[pl.ds(j * _L, _L)] for j in range(_NJ)]
            bet8 = [be_v[pl.ds(j * _L, _L)] for j in range(_NJ)]

            def allsum(v):
                for sh in (1, 2, 4, 8):
                    v = v + _take16(v, iota ^ sh)
                return v

            for rb in range(4):
                rows = [r0 + rb * 4 + r for r in range(4)]
                mrow = [(m_v[ihalf + rows[r], pl.ds(0, _L)],
                         m_v[ihalf + rows[r], pl.ds(_L, _L)])
                        for r in range(4)]
                acc = [[bias8[j] for j in range(_NJ)] for _ in range(4)]
                for k in range(_NUM_BET_BINS):
                    wk = [w_v[k, pl.ds(j * _L, _L)] for j in range(_NJ)]
                    for r in range(4):
                        mk = mrow[r][k // _L][k % _L]
                        for j in range(_NJ):
                            acc[r][j] = acc[r][j] + wk[j] * mk

                for r in range(4):
                    lane = rb * 4 + r
                    h = acc[r]
                    tot = h[0]
                    for j in range(1, _NJ):
                        tot = tot + h[j]
                    mean = allsum(tot) * (1.0 / _D)
                    xc = [h[j] - mean for j in range(_NJ)]
                    s2 = xc[0] * xc[0]
                    for j in range(1, _NJ):
                        s2 = s2 + xc[j] * xc[j]
                    var = allsum(s2) * (1.0 / _D)
                    y = _rsqrt16(var + 1e-5)
                    cix = cix16[lane]
                    vscale = valid16[lane]
                    for j in range(_NJ):
                        hn = xc[j] * y * gam8[j] + bet8[j]
                        hn = jnp.maximum(hn, 0.0)
                        e = comb_v[cix, pl.ds(j * _L, _L)]
                        out_v[half + rows[r], pl.ds(j * _L, _L)] = (
                            (hn + e) * vscale)
            return gcarry

        lax.fori_loop(0, _C // _L, sg_body, 0)

        pltpu.async_copy(out_v.at[pl.ds(half, _C)],
                         out_hbm.at[pl.ds(row0, _C)], sem_o)
        return carry

    lax.fori_loop(0, _NCH, chunk_body, 0)

    for _ in range(2):
        pltpu.make_async_copy(out_v.at[pl.ds(0, _C)],
                              out_hbm.at[pl.ds(base, _C)], sem_o).wait()


def kernel(token_ids, action_actors, action_streets, action_legal_masks,
           actor_emb_w, street_emb_w, action_type_emb_w, mlp_w, mlp_b,
           ln_gamma, ln_beta):
    b, l = token_ids.shape
    n = b * l
    out = _sc_run(
        action_legal_masks.reshape(n, _NUM_BET_BINS),
        token_ids.reshape(n).astype(jnp.int32),
        action_actors.reshape(n).astype(jnp.int32),
        action_streets.reshape(n).astype(jnp.int32),
        actor_emb_w, street_emb_w, action_type_emb_w, mlp_w,
        mlp_b, ln_gamma, ln_beta)
    return out.reshape(b, l, _D)


# speedup vs baseline: 4.2299x; 4.2299x over previous
import jax
import jax.numpy as jnp
from jax import lax
from jax.experimental import pallas as pl
from jax.experimental.pallas import tpu as pltpu

_NUM_BET_BINS = 32
_D_MODEL = 128
_NUM_STREETS = 4
_ACTION_OFFSET = 10

_N = 204800
_BLK = 4096
_NQ = 4
_G = _N // _BLK


def _fused_kernel(tok_ref, act_ref, st_ref, masks_ref, table_ref, mlp_w_ref,
                  mlp_b_ref, gamma_ref, beta_ref, out_ref, scratch, sem):
    i = pl.program_id(0)
    q = i % _NQ
    slot = pl.multiple_of(q * _BLK, _BLK)

    @pl.when(i >= _NQ)
    def _reclaim():
        pltpu.make_async_copy(
            scratch.at[pl.ds(slot, _BLK)],
            out_ref.at[pl.ds((i - _NQ) * _BLK, _BLK)], sem.at[q]).wait()

    tok = tok_ref[...]
    act = act_ref[...]
    st = st_ref[...]

    r = tok.shape[0]
    valid = ((tok >= _ACTION_OFFSET)
             & (tok < _ACTION_OFFSET + _NUM_BET_BINS)).astype(jnp.float32)
    aid = jnp.clip(tok - _ACTION_OFFSET, 0, _NUM_BET_BINS - 1)
    act = jnp.clip(act, 0, 1)
    st = jnp.clip(st, 0, _NUM_STREETS - 1)

    i38 = lax.broadcasted_iota(jnp.int32, (r, 38), 1)
    oh = jnp.where(i38 < 2, (act == i38).astype(jnp.float32), 0.0)
    oh = jnp.where((i38 >= 2) & (i38 < 6),
                   (st == i38 - 2).astype(jnp.float32), oh)
    oh = jnp.where(i38 >= 6, (aid == i38 - 6).astype(jnp.float32), oh)

    emb = jnp.dot(oh, table_ref[...], preferred_element_type=jnp.float32)

    h = jnp.dot(masks_ref[...], mlp_w_ref[...],
                preferred_element_type=jnp.float32) + mlp_b_ref[...]
    m = jnp.mean(h, axis=1, keepdims=True)
    c = h - m
    v = jnp.mean(c * c, axis=1, keepdims=True)
    h = c * lax.rsqrt(v + 1e-5) * gamma_ref[...] + beta_ref[...]
    h = jnp.maximum(h, 0.0)

    scratch[pl.ds(slot, _BLK), :] = valid * (emb + h)

    pltpu.make_async_copy(
        scratch.at[pl.ds(slot, _BLK)],
        out_ref.at[pl.ds(i * _BLK, _BLK)], sem.at[q]).start()

    @pl.when(i == _G - 1)
    def _drain():
        for j in range(_NQ):
            step = _G - 1 - j
            qq = step % _NQ
            pltpu.make_async_copy(
                scratch.at[pl.ds(qq * _BLK, _BLK)],
                out_ref.at[pl.ds(step * _BLK, _BLK)], sem.at[qq]).wait()


@jax.jit
def _run(token_ids, action_actors, action_streets, action_legal_masks,
         table, mlp_w, mlp_b, ln_gamma, ln_beta):
    b, l = token_ids.shape
    n = b * l
    tok = token_ids.reshape(n, 1).astype(jnp.int32)
    act = action_actors.reshape(n, 1).astype(jnp.int32)
    st = action_streets.reshape(n, 1).astype(jnp.int32)
    masks = action_legal_masks.reshape(n, _NUM_BET_BINS)

    row_spec = pl.BlockSpec((_BLK, 1), lambda i: (i, 0))
    full = lambda shape: pl.BlockSpec(shape, lambda i: (0, 0))

    out = pl.pallas_call(
        _fused_kernel,
        grid=(_G,),
        in_specs=[
            row_spec, row_spec, row_spec,
            pl.BlockSpec((_BLK, _NUM_BET_BINS), lambda i: (i, 0)),
            full(table.shape),
            full(mlp_w.shape),
            full((1, _D_MODEL)),
            full((1, _D_MODEL)),
            full((1, _D_MODEL)),
        ],
        out_specs=pl.BlockSpec(memory_space=pltpu.MemorySpace.HBM),
        out_shape=jax.ShapeDtypeStruct((n, _D_MODEL), jnp.float32),
        scratch_shapes=[pltpu.VMEM((_NQ * _BLK, _D_MODEL), jnp.float32),
                        pltpu.SemaphoreType.DMA((_NQ,))],
    )(tok, act, st, masks, table, mlp_w,
      mlp_b.reshape(1, _D_MODEL), ln_gamma.reshape(1, _D_MODEL),
      ln_beta.reshape(1, _D_MODEL))
    return out.reshape(b, l, _D_MODEL)


def kernel(token_ids, action_actors, action_streets, action_legal_masks,
           actor_emb_w, street_emb_w, action_type_emb_w, mlp_w, mlp_b,
           ln_gamma, ln_beta):
    table = jnp.concatenate([actor_emb_w, street_emb_w, action_type_emb_w],
                            axis=0)
    return _run(token_ids, action_actors, action_streets, action_legal_masks,
                table, mlp_w, mlp_b, ln_gamma, ln_beta)
